# baseline (speedup 1.0000x reference)
import functools

import jax
import jax.numpy as jnp
from jax import lax
from jax.experimental import pallas as pl
from jax.experimental.pallas import tpu as pltpu
from jax.experimental.pallas import tpu_sc as plsc

EMBED = 64
ROWS = 16384
COLS = 50
NC = 2
NS = 16
NW = NC * NS
R_PER_W = ROWS // NW

CR = 8
NCHUNK = R_PER_W // CR
NPAIR = NCHUNK // 2

_mesh = plsc.VectorSubcoreMesh(core_axis_name="c", subcore_axis_name="s")


@functools.partial(
    pl.kernel,
    mesh=_mesh,
    out_type=jax.ShapeDtypeStruct((ROWS, COLS, EMBED), jnp.float32),
    scratch_types=[
        pltpu.VMEM((2, CR, COLS), jnp.int32),
        pltpu.VMEM((2, CR, COLS, EMBED), jnp.float32),
        pltpu.SemaphoreType.DMA,
        pltpu.SemaphoreType.DMA,
        pltpu.SemaphoreType.DMA,
        pltpu.SemaphoreType.DMA,
    ],
    compiler_params=pltpu.CompilerParams(use_tc_tiling_on_sc=False),
)
def _embed_sc(x_hbm, table_hbm, out_hbm, idx_v, rows_v, sem_g0, sem_g1,
              sem_w0, sem_w1):
    wid = lax.axis_index("s") * NC + lax.axis_index("c")
    base = wid * R_PER_W
    sem_g = (sem_g0, sem_g1)
    sem_w = (sem_w0, sem_w1)

    def load_idx(c, b):
        xr = pl.multiple_of(base + c * CR, CR)
        pltpu.sync_copy(x_hbm.at[pl.ds(xr, CR)], idx_v.at[b])

    def fire_gathers(b):
        for r in range(CR):
            pltpu.async_copy(
                table_hbm.at[idx_v.at[b, r]],
                rows_v.at[b, r],
                sem_g[b],
            )

    def wait_gathers(b):
        for r in range(CR):
            pltpu.make_async_copy(
                table_hbm.at[idx_v.at[b, r]],
                rows_v.at[b, r],
                sem_g[b],
            ).wait()

    def fire_writeback(c, b):
        xr = pl.multiple_of(base + c * CR, CR)
        pltpu.async_copy(rows_v.at[b], out_hbm.at[pl.ds(xr, CR)], sem_w[b])

    def wait_writeback(b):
        pltpu.make_async_copy(
            rows_v.at[b], out_hbm.at[pl.ds(0, CR)], sem_w[b]
        ).wait()

    load_idx(0, 0)
    fire_gathers(0)

    def body(g, carry):
        c0 = g * 2
        c1 = c0 + 1
        load_idx(c1, 1)

        @pl.when(g > 0)
        def _():
            wait_writeback(1)

        fire_gathers(1)
        wait_gathers(0)
        fire_writeback(c0, 0)
        @pl.when(g < NPAIR - 1)
        def _():
            load_idx(c0 + 2, 0)
            wait_writeback(0)
            fire_gathers(0)

        wait_gathers(1)
        fire_writeback(c1, 1)
        return carry

    lax.fori_loop(0, NPAIR, body, 0)
    wait_writeback(0)
    wait_writeback(1)


def kernel(x, table):
    return _embed_sc(x.astype(jnp.int32), table)
